# baseline (speedup 1.0000x reference)
import functools

import jax
import jax.numpy as jnp
from jax import lax
from jax.experimental import pallas as pl
from jax.experimental.pallas import tpu as pltpu
from jax.experimental.pallas import tpu_sc as plsc

_B, _S = 16384, 50
_D = 32
_SP, _DP = 56, 128
_V = 1000000
_PK = _V // 4

_info = plsc.get_sparse_core_info()
_NC, _NS = _info.num_cores, _info.num_subcores
_NW = _NC * _NS
_L = 16

_ROWS_PER_W = _B // _NW
_KB = 8
_STEPS_B = _ROWS_PER_W // _KB


def _gather_body(idx, wp, out, idx_v, pid_v, sub_v, g_v, o_v, sem):
    wid = lax.axis_index("s") * _NC + lax.axis_index("c")
    base = wid * _ROWS_PER_W

    def step(g, carry):
        row_off = base + g * _KB
        pltpu.sync_copy(idx.at[pl.ds(row_off, _KB)], idx_v)
        for j in range(_KB):
            for a in range(4):
                v = idx_v[j, pl.ds(a * _L, _L)]
                pid_v[j, pl.ds(a * _L, _L)] = lax.shift_right_logical(v, 2)
                sub_v[j, pl.ds(a * _L, _L)] = lax.shift_left(
                    jnp.bitwise_and(v, 3), 5
                )
        copies = [
            pltpu.async_copy(
                wp.at[pid_v.at[j, pl.ds(0, _S)]], g_v.at[j], sem
            )
            for j in range(_KB)
        ]
        for c in copies:
            c.wait()

        def extract(t, carry2):
            for j in range(_KB):
                sub = sub_v[j, pl.ds(t, _L)][0]
                for h in range(2):
                    o_v[j, t, pl.ds(h * _L, _L)] = (
                        g_v[j, t, pl.ds(sub + h * _L, _L)]
                    )
            return carry2

        lax.fori_loop(0, _S, extract, 0)
        pltpu.sync_copy(o_v, out.at[pl.ds(row_off, _KB)])
        return carry

    lax.fori_loop(0, _STEPS_B, step, 0)


@functools.partial(
    pl.kernel,
    mesh=plsc.VectorSubcoreMesh(core_axis_name="c", subcore_axis_name="s"),
    out_type=jax.ShapeDtypeStruct((_B, _SP, _DP), jnp.float32),
    scratch_types=[
        pltpu.VMEM((_KB, _DP), jnp.int32),
        pltpu.VMEM((_KB, 64), jnp.int32),
        pltpu.VMEM((_KB, 80), jnp.int32),
        pltpu.VMEM((_KB, _S, _DP), jnp.float32),
        pltpu.VMEM((_KB, _SP, _DP), jnp.float32),
        pltpu.SemaphoreType.DMA,
    ],
)
def _gather_kernel(idx, wp, out, idx_v, pid_v, sub_v, g_v, o_v, sem):
    _gather_body(idx, wp, out, idx_v, pid_v, sub_v, g_v, o_v, sem)


def kernel(token_ids, weight):
    ids = jnp.pad(token_ids.astype(jnp.int32), ((0, 0), (0, _DP - _S)))
    wp = weight.reshape(_PK, _DP)
    out = _gather_kernel(ids, wp)
    return out[:, :_S, :_D]


# speedup vs baseline: 2.0789x; 2.0789x over previous
import functools

import jax
import jax.numpy as jnp
from jax import lax
from jax.experimental import pallas as pl
from jax.experimental.pallas import tpu as pltpu
from jax.experimental.pallas import tpu_sc as plsc

_B, _S = 16384, 50
_D = 32
_SP, _DP = 56, 128

_info = plsc.get_sparse_core_info()
_NC, _NS = _info.num_cores, _info.num_subcores
_NW = _NC * _NS

_ROWS_PER_W = _B // _NW
_K = 16
_N_STEPS = _ROWS_PER_W // _K


def _emb_body(idx, table, out, idx_v, rows_v, sem):
    wid = lax.axis_index("s") * _NC + lax.axis_index("c")
    base = wid * _ROWS_PER_W

    def step(g, carry):
        row_off = base + g * _K
        pltpu.sync_copy(idx.at[pl.ds(row_off, _K)], idx_v)
        copies = [
            pltpu.async_copy(table.at[idx_v.at[j]], rows_v.at[j], sem)
            for j in range(_K)
        ]
        for c in copies:
            c.wait()
        pltpu.sync_copy(
            rows_v,
            out.at[pl.ds(row_off, _K), pl.ds(0, _S), pl.ds(0, _D)],
        )
        return carry

    lax.fori_loop(0, _N_STEPS, step, 0)


@functools.partial(
    pl.kernel,
    mesh=plsc.VectorSubcoreMesh(core_axis_name="c", subcore_axis_name="s"),
    out_type=jax.ShapeDtypeStruct((_B, _SP, _DP), jnp.float32),
    scratch_types=[
        pltpu.VMEM((_K, _S), jnp.int32),
        pltpu.VMEM((_K, _S, _D), jnp.float32),
        pltpu.SemaphoreType.DMA,
    ],
    compiler_params=pltpu.CompilerParams(use_tc_tiling_on_sc=False),
)
def _emb_kernel(idx, table, out, idx_v, rows_v, sem):
    _emb_body(idx, table, out, idx_v, rows_v, sem)


def kernel(token_ids, weight):
    out = _emb_kernel(token_ids.astype(jnp.int32), weight)
    return out[:, :_S, :_D]


# speedup vs baseline: 2.1423x; 1.0305x over previous
import functools

import jax
import jax.numpy as jnp
from jax import lax
from jax.experimental import pallas as pl
from jax.experimental.pallas import tpu as pltpu
from jax.experimental.pallas import tpu_sc as plsc

_B, _S = 16384, 50
_D = 32
_SP, _DP = 56, 128

_info = plsc.get_sparse_core_info()
_NC, _NS = _info.num_cores, _info.num_subcores
_NW = _NC * _NS

_ROWS_PER_W = _B // _NW
_K = 32
_N_STEPS = _ROWS_PER_W // _K


def _emb_body(idx, table, out, idx_v, rows_v, sem):
    wid = lax.axis_index("s") * _NC + lax.axis_index("c")
    base = wid * _ROWS_PER_W

    def step(g, carry):
        row_off = base + g * _K
        pltpu.sync_copy(idx.at[pl.ds(row_off, _K)], idx_v)
        copies = [
            pltpu.async_copy(table.at[idx_v.at[j]], rows_v.at[j], sem)
            for j in range(_K)
        ]
        for c in copies:
            c.wait()
        pltpu.sync_copy(
            rows_v,
            out.at[pl.ds(row_off, _K), pl.ds(0, _S), pl.ds(0, _D)],
        )
        return carry

    lax.fori_loop(0, _N_STEPS, step, 0)


@functools.partial(
    pl.kernel,
    mesh=plsc.VectorSubcoreMesh(core_axis_name="c", subcore_axis_name="s"),
    out_type=jax.ShapeDtypeStruct((_B, _SP, _DP), jnp.float32),
    scratch_types=[
        pltpu.VMEM((_K, _S), jnp.int32),
        pltpu.VMEM((_K, _S, _D), jnp.float32),
        pltpu.SemaphoreType.DMA,
    ],
    compiler_params=pltpu.CompilerParams(use_tc_tiling_on_sc=False),
)
def _emb_kernel(idx, table, out, idx_v, rows_v, sem):
    _emb_body(idx, table, out, idx_v, rows_v, sem)


def kernel(token_ids, weight):
    out = _emb_kernel(token_ids.astype(jnp.int32), weight)
    return out[:, :_S, :_D]


# speedup vs baseline: 2.1777x; 1.0166x over previous
import functools

import jax
import jax.numpy as jnp
from jax import lax
from jax.experimental import pallas as pl
from jax.experimental.pallas import tpu as pltpu
from jax.experimental.pallas import tpu_sc as plsc

_B, _S = 16384, 50
_D = 32
_SP, _DP = 56, 128

_info = plsc.get_sparse_core_info()
_NC, _NS = _info.num_cores, _info.num_subcores
_NW = _NC * _NS

_ROWS_PER_W = _B // _NW
_K = 64
_N_STEPS = _ROWS_PER_W // _K


def _emb_body(idx, table, out, idx_v, rows_v, sem):
    wid = lax.axis_index("s") * _NC + lax.axis_index("c")
    base = wid * _ROWS_PER_W

    def step(g, carry):
        row_off = base + g * _K
        pltpu.sync_copy(idx.at[pl.ds(row_off, _K)], idx_v)
        copies = [
            pltpu.async_copy(table.at[idx_v.at[j]], rows_v.at[j], sem)
            for j in range(_K)
        ]
        for c in copies:
            c.wait()
        pltpu.sync_copy(
            rows_v,
            out.at[pl.ds(row_off, _K), pl.ds(0, _S), pl.ds(0, _D)],
        )
        return carry

    lax.fori_loop(0, _N_STEPS, step, 0)


@functools.partial(
    pl.kernel,
    mesh=plsc.VectorSubcoreMesh(core_axis_name="c", subcore_axis_name="s"),
    out_type=jax.ShapeDtypeStruct((_B, _SP, _DP), jnp.float32),
    scratch_types=[
        pltpu.VMEM((_K, _S), jnp.int32),
        pltpu.VMEM((_K, _S, _D), jnp.float32),
        pltpu.SemaphoreType.DMA,
    ],
    compiler_params=pltpu.CompilerParams(use_tc_tiling_on_sc=False),
)
def _emb_kernel(idx, table, out, idx_v, rows_v, sem):
    _emb_body(idx, table, out, idx_v, rows_v, sem)


def kernel(token_ids, weight):
    out = _emb_kernel(token_ids.astype(jnp.int32), weight)
    return out[:, :_S, :_D]
